# baseline (speedup 1.0000x reference)
import functools

import jax
import jax.numpy as jnp
from jax import lax
from jax.experimental import pallas as pl
from jax.experimental.pallas import tpu as pltpu
from jax.experimental.pallas import tpu_sc as plsc

M_ROWS = 1000000
DIM = 16
LANES = 16
NUM_CORES = 2
NUM_SUBCORES = 16
NUM_WORKERS = NUM_CORES * NUM_SUBCORES

CHUNK_ROWS = 800
GROUPS = CHUNK_ROWS // LANES
NUM_CHUNKS = M_ROWS // CHUNK_ROWS


def _body(items_hbm, ubc_hbm, out_hbm, ubuf, chunk_v, out_v):
    wid = lax.axis_index("s") * NUM_CORES + lax.axis_index("c")

    pltpu.sync_copy(ubc_hbm, ubuf)
    u_bcast = [ubuf[d] for d in range(DIM)]
    col_idx = [jnp.full((LANES,), d, jnp.int32) for d in range(DIM)]
    lane_iota = lax.iota(jnp.int32, LANES)

    def group_body(g, _):
        rows = jnp.full((LANES,), g * LANES, jnp.int32) + lane_iota
        acc = plsc.load_gather(chunk_v, [rows, col_idx[0]]) * u_bcast[0]
        for d in range(1, DIM):
            acc = acc + plsc.load_gather(chunk_v, [rows, col_idx[d]]) * u_bcast[d]
        out_v[pl.ds(g * LANES, LANES)] = acc
        return 0

    def chunk_body(i, _):
        base = (wid + i * NUM_WORKERS) * CHUNK_ROWS
        pltpu.sync_copy(items_hbm.at[pl.ds(base, CHUNK_ROWS)], chunk_v)
        lax.fori_loop(0, GROUPS, group_body, 0, unroll=False)
        pltpu.sync_copy(out_v, out_hbm.at[pl.ds(base, CHUNK_ROWS)])
        return 0

    ncw = (NUM_CHUNKS - wid + NUM_WORKERS - 1) // NUM_WORKERS
    lax.fori_loop(0, ncw, chunk_body, 0, unroll=False)


def kernel(items_emb, user_emb):
    mesh = plsc.VectorSubcoreMesh(
        core_axis_name="c", subcore_axis_name="s",
        num_cores=NUM_CORES, num_subcores=NUM_SUBCORES,
    )
    run = pl.kernel(
        _body,
        out_type=jax.ShapeDtypeStruct((M_ROWS,), jnp.float32),
        mesh=mesh,
        compiler_params=pltpu.CompilerParams(needs_layout_passes=False),
        scratch_types=[
            pltpu.VMEM((DIM, LANES), jnp.float32),
            pltpu.VMEM((CHUNK_ROWS, DIM), jnp.float32),
            pltpu.VMEM((CHUNK_ROWS,), jnp.float32),
        ],
    )
    u_bcast_table = jnp.broadcast_to(
        user_emb.reshape(DIM, 1), (DIM, LANES)
    )
    return run(items_emb, u_bcast_table)


# speedup vs baseline: 1.3331x; 1.3331x over previous
import functools

import jax
import jax.numpy as jnp
from jax import lax
from jax.experimental import pallas as pl
from jax.experimental.pallas import tpu as pltpu
from jax.experimental.pallas import tpu_sc as plsc

M_ROWS = 1000000
DIM = 16
LANES = 16
NUM_CORES = 2
NUM_SUBCORES = 16
NUM_WORKERS = NUM_CORES * NUM_SUBCORES

CHUNK_ROWS = 800
GROUPS = CHUNK_ROWS // LANES
NUM_CHUNKS = M_ROWS // CHUNK_ROWS


def _body(items_hbm, user_hbm, out_hbm, ubuf, chunk_v, out_v):
    wid = lax.axis_index("s") * NUM_CORES + lax.axis_index("c")

    pltpu.sync_copy(user_hbm.at[0], ubuf)
    u = ubuf[...]

    lane = lax.iota(jnp.int32, LANES)
    perm_idx = [lane ^ k for k in (1, 2, 4, 8)]
    masks = [(lane & k) == 0 for k in (1, 2, 4, 8)]

    def rowsum16(vecs):
        for lvl in range(4):
            m, p = masks[lvl], perm_idx[lvl]
            nxt = []
            for i in range(0, len(vecs), 2):
                a, b = vecs[i], vecs[i + 1]
                s = jnp.where(m, a, b)
                t = jnp.where(m, b, a)
                nxt.append(
                    s + jnp.take_along_axis(t, p, 0, mode="promise_in_bounds")
                )
            vecs = nxt
        return vecs[0]

    def group_body(g, _):
        base = g * LANES
        prods = [chunk_v[base + j] * u for j in range(LANES)]
        out_v[pl.ds(g * LANES, LANES)] = rowsum16(prods)
        return 0

    def chunk_body(i, _):
        base = (wid + i * NUM_WORKERS) * CHUNK_ROWS
        pltpu.sync_copy(items_hbm.at[pl.ds(base, CHUNK_ROWS)], chunk_v)
        lax.fori_loop(0, GROUPS, group_body, 0, unroll=False)
        pltpu.sync_copy(out_v, out_hbm.at[pl.ds(base, CHUNK_ROWS)])
        return 0

    ncw = (NUM_CHUNKS - wid + NUM_WORKERS - 1) // NUM_WORKERS
    lax.fori_loop(0, ncw, chunk_body, 0, unroll=False)


def kernel(items_emb, user_emb):
    mesh = plsc.VectorSubcoreMesh(
        core_axis_name="c", subcore_axis_name="s",
        num_cores=NUM_CORES, num_subcores=NUM_SUBCORES,
    )
    run = pl.kernel(
        _body,
        out_type=jax.ShapeDtypeStruct((M_ROWS,), jnp.float32),
        mesh=mesh,
        compiler_params=pltpu.CompilerParams(needs_layout_passes=False),
        scratch_types=[
            pltpu.VMEM((DIM,), jnp.float32),
            pltpu.VMEM((CHUNK_ROWS, DIM), jnp.float32),
            pltpu.VMEM((CHUNK_ROWS,), jnp.float32),
        ],
    )
    return run(items_emb, user_emb)


# speedup vs baseline: 5.8727x; 4.4054x over previous
import functools

import jax
import jax.numpy as jnp
from jax import lax
from jax.experimental import pallas as pl
from jax.experimental.pallas import tpu as pltpu
from jax.experimental.pallas import tpu_sc as plsc

M_ROWS = 1000000
DIM = 16
LANES = 16
NUM_CORES = 2
NUM_SUBCORES = 16
NUM_WORKERS = NUM_CORES * NUM_SUBCORES

CHUNK_COLS = 1536
GROUPS = CHUNK_COLS // LANES
BULK_ROWS = 999936
NUM_CHUNKS = BULK_ROWS // CHUNK_COLS
TAIL = M_ROWS - BULK_ROWS


def _body(itT_hbm, tail_hbm, ubc_hbm, out_hbm,
          ub_v, bufA, bufB, out_v, tail_in, tail_out):
    wid = lax.axis_index("s") * NUM_CORES + lax.axis_index("c")

    pltpu.sync_copy(ubc_hbm, ub_v)
    ub = [ub_v[d] for d in range(DIM)]

    def group_body(g, _):
        off = g * LANES
        acc = bufA[0, pl.ds(off, LANES)] * ub[0]
        for d in range(1, 8):
            acc = acc + bufA[d, pl.ds(off, LANES)] * ub[d]
        for d in range(8, DIM):
            acc = acc + bufB[d - 8, pl.ds(off, LANES)] * ub[d]
        out_v[pl.ds(off, LANES)] = acc
        return 0

    def chunk_body(i, _):
        col0 = (wid + i * NUM_WORKERS) * CHUNK_COLS
        pltpu.sync_copy(itT_hbm.at[pl.ds(0, 8), pl.ds(col0, CHUNK_COLS)], bufA)
        pltpu.sync_copy(itT_hbm.at[pl.ds(8, 8), pl.ds(col0, CHUNK_COLS)], bufB)
        lax.fori_loop(0, GROUPS, group_body, 0, unroll=False)
        pltpu.sync_copy(out_v, out_hbm.at[pl.ds(col0, CHUNK_COLS)])
        return 0

    ncw = 20 + (wid < 11).astype(jnp.int32)
    lax.fori_loop(0, ncw, chunk_body, 0, unroll=False)

    @pl.when(wid == NUM_WORKERS - 1)
    def _():
        lane = lax.iota(jnp.int32, LANES)
        u = plsc.load_gather(ub_v, [lane, lane])
        pltpu.sync_copy(tail_hbm, tail_in)
        perms = [lane ^ k for k in (1, 2, 4, 8)]
        masks = [(lane & k) == 0 for k in (1, 2, 4, 8)]
        for g in range(TAIL // LANES):
            vecs = [tail_in[g * LANES + j] * u for j in range(LANES)]
            for lvl in range(4):
                m, p = masks[lvl], perms[lvl]
                nxt = []
                for i in range(0, len(vecs), 2):
                    a, b = vecs[i], vecs[i + 1]
                    s = jnp.where(m, a, b)
                    t = jnp.where(m, b, a)
                    nxt.append(
                        s + jnp.take_along_axis(t, p, 0,
                                                mode="promise_in_bounds")
                    )
                vecs = nxt
            tail_out[pl.ds(g * LANES, LANES)] = vecs[0]
        pltpu.sync_copy(tail_out, out_hbm.at[pl.ds(BULK_ROWS, TAIL)])


def kernel(items_emb, user_emb):
    mesh = plsc.VectorSubcoreMesh(
        core_axis_name="c", subcore_axis_name="s",
        num_cores=NUM_CORES, num_subcores=NUM_SUBCORES,
    )
    run = pl.kernel(
        _body,
        out_type=jax.ShapeDtypeStruct((M_ROWS,), jnp.float32),
        mesh=mesh,
        compiler_params=pltpu.CompilerParams(
            needs_layout_passes=False, use_tc_tiling_on_sc=True,
        ),
        scratch_types=[
            pltpu.VMEM((DIM, LANES), jnp.float32),
            pltpu.VMEM((8, CHUNK_COLS), jnp.float32),
            pltpu.VMEM((8, CHUNK_COLS), jnp.float32),
            pltpu.VMEM((CHUNK_COLS,), jnp.float32),
            pltpu.VMEM((TAIL, DIM), jnp.float32),
            pltpu.VMEM((TAIL,), jnp.float32),
        ],
    )
    items_t = items_emb.T
    tail_rows = items_emb[BULK_ROWS:]
    u_bcast_table = jnp.broadcast_to(user_emb.reshape(DIM, 1), (DIM, LANES))
    return run(items_t, tail_rows, u_bcast_table)


# speedup vs baseline: 9.7903x; 1.6671x over previous
import functools

import jax
import jax.numpy as jnp
from jax import lax
from jax.experimental import pallas as pl
from jax.experimental.pallas import tpu as pltpu
from jax.experimental.pallas import tpu_sc as plsc

M_ROWS = 1000000
DIM = 16
LANES = 16
NUM_CORES = 2
NUM_SUBCORES = 16
NUM_WORKERS = NUM_CORES * NUM_SUBCORES

CHUNK_COLS = 1536
GROUPS = CHUNK_COLS // LANES
BULK_ROWS = 999936
NUM_CHUNKS = BULK_ROWS // CHUNK_COLS
TAIL = M_ROWS - BULK_ROWS


def _body(itT_hbm, tail_hbm, ubc_hbm, out_hbm,
          ub_v, bufA0, bufB0, bufA1, bufB1, out0, out1, tail_in, tail_out,
          insem0, insem1, outsem0, outsem1):
    wid = lax.axis_index("s") * NUM_CORES + lax.axis_index("c")

    pltpu.sync_copy(ubc_hbm, ub_v)
    ub = [ub_v[d] for d in range(DIM)]

    def make_compute(bufA, bufB, out_v):
        def group_body(g, _):
            off = g * LANES
            acc = bufA[0, pl.ds(off, LANES)] * ub[0]
            for d in range(1, 8):
                acc = acc + bufA[d, pl.ds(off, LANES)] * ub[d]
            for d in range(8, DIM):
                acc = acc + bufB[d - 8, pl.ds(off, LANES)] * ub[d]
            out_v[pl.ds(off, LANES)] = acc
            return 0
        return lambda: lax.fori_loop(0, GROUPS, group_body, 0, unroll=False)

    compute0 = make_compute(bufA0, bufB0, out0)
    compute1 = make_compute(bufA1, bufB1, out1)

    def in_slices(c):
        col0 = c * CHUNK_COLS
        return (itT_hbm.at[pl.ds(0, 8), pl.ds(col0, CHUNK_COLS)],
                itT_hbm.at[pl.ds(8, 8), pl.ds(col0, CHUNK_COLS)])

    def start_in(c, bufA, bufB, sem):
        sa, sb = in_slices(c)
        pltpu.async_copy(sa, bufA, sem)
        pltpu.async_copy(sb, bufB, sem)

    def wait_in(c, bufA, bufB, sem):
        sa, sb = in_slices(c)
        pltpu.make_async_copy(sa, bufA, sem).wait()
        pltpu.make_async_copy(sb, bufB, sem).wait()

    def start_out(c, out_v, sem):
        pltpu.async_copy(out_v, out_hbm.at[pl.ds(c * CHUNK_COLS, CHUNK_COLS)],
                         sem)

    def wait_out(c, out_v, sem):
        pltpu.make_async_copy(
            out_v, out_hbm.at[pl.ds(c * CHUNK_COLS, CHUNK_COLS)], sem).wait()

    ncw = 20 + (wid < 11).astype(jnp.int32)
    cid = lambda i: wid + i * NUM_WORKERS

    start_in(cid(0), bufA0, bufB0, insem0)
    npairs = (ncw + 1) // 2

    def pair_body(p, _):
        j = 2 * p

        @pl.when(j + 1 < ncw)
        def _():
            start_in(cid(j + 1), bufA1, bufB1, insem1)

        wait_in(cid(j), bufA0, bufB0, insem0)

        @pl.when(p > 0)
        def _():
            wait_out(cid(j - 2), out0, outsem0)

        compute0()
        start_out(cid(j), out0, outsem0)

        @pl.when(j + 2 < ncw)
        def _():
            start_in(cid(j + 2), bufA0, bufB0, insem0)

        @pl.when(j + 1 < ncw)
        def _():
            wait_in(cid(j + 1), bufA1, bufB1, insem1)

            @pl.when(p > 0)
            def _():
                wait_out(cid(j - 1), out1, outsem1)

            compute1()
            start_out(cid(j + 1), out1, outsem1)

        return 0

    lax.fori_loop(0, npairs, pair_body, 0, unroll=False)
    wait_out(0, out0, outsem0)
    wait_out(0, out1, outsem1)

    @pl.when(wid == NUM_WORKERS - 1)
    def _():
        lane = lax.iota(jnp.int32, LANES)
        u = plsc.load_gather(ub_v, [lane, lane])
        pltpu.sync_copy(tail_hbm, tail_in)
        perms = [lane ^ k for k in (1, 2, 4, 8)]
        masks = [(lane & k) == 0 for k in (1, 2, 4, 8)]
        for g in range(TAIL // LANES):
            vecs = [tail_in[g * LANES + j] * u for j in range(LANES)]
            for lvl in range(4):
                m, p = masks[lvl], perms[lvl]
                nxt = []
                for i in range(0, len(vecs), 2):
                    a, b = vecs[i], vecs[i + 1]
                    s = jnp.where(m, a, b)
                    t = jnp.where(m, b, a)
                    nxt.append(
                        s + jnp.take_along_axis(t, p, 0,
                                                mode="promise_in_bounds")
                    )
                vecs = nxt
            tail_out[pl.ds(g * LANES, LANES)] = vecs[0]
        pltpu.sync_copy(tail_out, out_hbm.at[pl.ds(BULK_ROWS, TAIL)])


def kernel(items_emb, user_emb):
    mesh = plsc.VectorSubcoreMesh(
        core_axis_name="c", subcore_axis_name="s",
        num_cores=NUM_CORES, num_subcores=NUM_SUBCORES,
    )
    run = pl.kernel(
        _body,
        out_type=jax.ShapeDtypeStruct((M_ROWS,), jnp.float32),
        mesh=mesh,
        compiler_params=pltpu.CompilerParams(
            needs_layout_passes=False, use_tc_tiling_on_sc=True,
        ),
        scratch_types=[
            pltpu.VMEM((DIM, LANES), jnp.float32),
            pltpu.VMEM((8, CHUNK_COLS), jnp.float32),
            pltpu.VMEM((8, CHUNK_COLS), jnp.float32),
            pltpu.VMEM((8, CHUNK_COLS), jnp.float32),
            pltpu.VMEM((8, CHUNK_COLS), jnp.float32),
            pltpu.VMEM((CHUNK_COLS,), jnp.float32),
            pltpu.VMEM((CHUNK_COLS,), jnp.float32),
            pltpu.VMEM((TAIL, DIM), jnp.float32),
            pltpu.VMEM((TAIL,), jnp.float32),
            pltpu.SemaphoreType.DMA,
            pltpu.SemaphoreType.DMA,
            pltpu.SemaphoreType.DMA,
            pltpu.SemaphoreType.DMA,
        ],
    )
    items_t = items_emb.T
    tail_rows = items_emb[BULK_ROWS:]
    u_bcast_table = jnp.broadcast_to(user_emb.reshape(DIM, 1), (DIM, LANES))
    return run(items_t, tail_rows, u_bcast_table)
